# Initial kernel scaffold; baseline (speedup 1.0000x reference)
import jax
import jax.numpy as jnp
from jax.experimental import pallas as pl


def kernel(x, edge_index, ln0_g, ln0_b, w0l, b0l, w0r, ln1_g, ln1_b, w1l, b1l, w1r):
    raise NotImplementedError("write your pallas kernel here")


# speedup vs baseline: 4.8864x; 4.8864x over previous
import functools

import jax
import jax.numpy as jnp
from jax import lax
from jax.experimental import pallas as pl
from jax.experimental.pallas import tpu as pltpu
from jax.experimental.pallas import tpu_sc as plsc

N = 10000
E = 320000
D = 128

NC = 2
NS = 16
NW = NC * NS
EPW = E // NW
K = 80
NCHUNK = EPW // K
NP = 10240
RPT = NP // NS
ZR = 128
CW = 16

@functools.lru_cache(maxsize=None)
def _make_sc_agg(with_cnt: bool):
    out_type = [jax.ShapeDtypeStruct((NC, NP, D), jnp.float32)]
    scratch = [
        pltpu.VMEM((K,), jnp.int32),
        pltpu.VMEM((K,), jnp.int32),
        pltpu.VMEM((K, D), jnp.float32),
        pltpu.VMEM((ZR, D), jnp.float32),
        pltpu.VMEM_SHARED((NP, D), jnp.float32),
        pltpu.SemaphoreType.DMA,
    ]
    if with_cnt:
        out_type.append(jax.ShapeDtypeStruct((NC, NP, CW), jnp.float32))
        scratch += [
            pltpu.VMEM((K, CW), jnp.float32),
            pltpu.VMEM((RPT, CW), jnp.float32),
            pltpu.VMEM_SHARED((NP, CW), jnp.float32),
        ]

    def body(src_hbm, dst_hbm, y_hbm, z_hbm, *rest):
        if with_cnt:
            (ones_hbm, zc_hbm, agg_out, cnt_out,
             src_v, dst_v, rows_v, zbuf, agg_sh, sem,
             ones_v, cbuf, cnt_sh) = rest
        else:
            (agg_out, src_v, dst_v, rows_v, zbuf, agg_sh, sem) = rest
        c = lax.axis_index("c")
        s = lax.axis_index("s")
        wid = c * NS + s

        pltpu.sync_copy(z_hbm.at[pl.ds(0, ZR)], zbuf)
        for k in range(RPT // ZR):
            pltpu.sync_copy(zbuf, agg_sh.at[pl.ds(s * RPT + k * ZR, ZR)])
        if with_cnt:
            pltpu.sync_copy(ones_hbm.at[pl.ds(0, K)], ones_v)
            pltpu.sync_copy(zc_hbm.at[pl.ds(0, RPT)], cbuf)
            pltpu.sync_copy(cbuf, cnt_sh.at[pl.ds(s * RPT, RPT)])
        plsc.subcore_barrier()

        ebase = wid * EPW

        def chunk(i, carry):
            off = ebase + i * K
            pltpu.sync_copy(src_hbm.at[pl.ds(off, K)], src_v)
            pltpu.sync_copy(dst_hbm.at[pl.ds(off, K)], dst_v)
            pltpu.async_copy(y_hbm.at[src_v], rows_v, sem).wait()
            pltpu.sync_copy(rows_v, agg_sh.at[dst_v], add=True)
            if with_cnt:
                pltpu.sync_copy(ones_v, cnt_sh.at[dst_v], add=True)
            return carry

        lax.fori_loop(0, NCHUNK, chunk, 0)
        plsc.subcore_barrier()

        for k in range(RPT // ZR):
            r0 = s * RPT + k * ZR
            pltpu.sync_copy(agg_sh.at[pl.ds(r0, ZR)], zbuf)
            pltpu.sync_copy(zbuf, agg_out.at[c, pl.ds(r0, ZR)])
        if with_cnt:
            pltpu.sync_copy(cnt_sh.at[pl.ds(s * RPT, RPT)], cbuf)
            pltpu.sync_copy(cbuf, cnt_out.at[c, pl.ds(s * RPT, RPT)])

    ot = out_type if with_cnt else out_type[0]
    mesh = plsc.VectorSubcoreMesh(core_axis_name="c", subcore_axis_name="s")
    return pl.kernel(
        body, mesh=mesh, out_type=ot, scratch_types=scratch,
        compiler_params=pltpu.CompilerParams(use_tc_tiling_on_sc=False))


BR = 1000


def _ln_relu(x, g, b):
    m = jnp.mean(x, axis=-1, keepdims=True)
    v = jnp.mean((x - m) ** 2, axis=-1, keepdims=True)
    h = (x - m) * lax.rsqrt(v + 1e-5) * g + b
    return jnp.maximum(h, 0.0)


def _tc_pre_body(x_ref, g_ref, b_ref, wlt_ref, wrt_ref, y_ref, s_ref):
    h = _ln_relu(x_ref[...], g_ref[...], b_ref[...])
    y_ref[...] = jnp.dot(h, wlt_ref[...], preferred_element_type=jnp.float32)
    s_ref[...] = jnp.dot(h, wrt_ref[...], preferred_element_type=jnp.float32)


def _mean_from_partials(agg, cnt):
    a = agg[0] + agg[1]
    c = cnt[0, :, 0:1] + cnt[1, :, 0:1]
    return a / jnp.maximum(c, 1.0)


def _tc_mid_body(agg_ref, cnt_ref, s0_ref, b0l_ref, g_ref, b_ref,
                 wlt_ref, wrt_ref, y_ref, s_ref):
    t = _mean_from_partials(agg_ref[...], cnt_ref[...]) + b0l_ref[...] + s0_ref[...]
    h = _ln_relu(t, g_ref[...], b_ref[...])
    y_ref[...] = jnp.dot(h, wlt_ref[...], preferred_element_type=jnp.float32)
    s_ref[...] = jnp.dot(h, wrt_ref[...], preferred_element_type=jnp.float32)


def _tc_post_body(x_ref, agg_ref, cnt_ref, s1_ref, b1l_ref, out_ref):
    t = _mean_from_partials(agg_ref[...], cnt_ref[...]) + b1l_ref[...] + s1_ref[...]
    out_ref[...] = x_ref[...] + t


_row_spec = pl.BlockSpec((BR, D), lambda i: (i, 0))
_vec_spec = pl.BlockSpec((1, D), lambda i: (0, 0))
_w_spec = pl.BlockSpec((D, D), lambda i: (0, 0))
_agg_spec = pl.BlockSpec((NC, BR, D), lambda i: (0, i, 0))
_cnt_spec = pl.BlockSpec((NC, BR, CW), lambda i: (0, i, 0))
_GRID = (N // BR,)

_tc_pre = pl.pallas_call(
    _tc_pre_body,
    grid=_GRID,
    in_specs=[_row_spec, _vec_spec, _vec_spec, _w_spec, _w_spec],
    out_specs=[_row_spec, _row_spec],
    out_shape=[jax.ShapeDtypeStruct((N, D), jnp.float32)] * 2,
)

_tc_mid = pl.pallas_call(
    _tc_mid_body,
    grid=_GRID,
    in_specs=[_agg_spec, _cnt_spec, _row_spec, _vec_spec, _vec_spec,
              _vec_spec, _w_spec, _w_spec],
    out_specs=[_row_spec, _row_spec],
    out_shape=[jax.ShapeDtypeStruct((N, D), jnp.float32)] * 2,
)

_tc_post = pl.pallas_call(
    _tc_post_body,
    grid=_GRID,
    in_specs=[_row_spec, _agg_spec, _cnt_spec, _row_spec, _vec_spec],
    out_specs=_row_spec,
    out_shape=jax.ShapeDtypeStruct((N, D), jnp.float32),
)


def kernel(x, edge_index, ln0_g, ln0_b, w0l, b0l, w0r, ln1_g, ln1_b,
           w1l, b1l, w1r):
    src = edge_index[0]
    dst = edge_index[1]
    g0 = ln0_g.reshape(1, D)
    b0 = ln0_b.reshape(1, D)
    g1 = ln1_g.reshape(1, D)
    b1 = ln1_b.reshape(1, D)
    b0l2 = b0l.reshape(1, D)
    b1l2 = b1l.reshape(1, D)
    zrows = jnp.zeros((ZR, D), jnp.float32)
    zcnt = jnp.zeros((RPT, CW), jnp.float32)
    ones = jnp.ones((K, CW), jnp.float32)

    y0, s0 = _tc_pre(x, g0, b0, w0l.T, w0r.T)
    agg0, cnt = _make_sc_agg(True)(src, dst, y0, zrows, ones, zcnt)
    y1, s1 = _tc_mid(agg0, cnt, s0, b0l2, g1, b1, w1l.T, w1r.T)
    agg1 = _make_sc_agg(False)(src, dst, y1, zrows)
    return _tc_post(x, agg1, cnt, s1, b1l2)


# speedup vs baseline: 9.1117x; 1.8647x over previous
import functools

import jax
import jax.numpy as jnp
from jax import lax
from jax.experimental import pallas as pl
from jax.experimental.pallas import tpu as pltpu
from jax.experimental.pallas import tpu_sc as plsc

N = 10000
E = 320000
D = 128

NC = 2
NS = 16
NW = NC * NS
EPW = E // NW
K = 80
NCHUNK = EPW // K
NPAIR = NCHUNK // 2
NP = 10240
RPT = NP // NS


@functools.lru_cache(maxsize=None)
def _make_sc_agg(with_cnt: bool):
    out_type = [jax.ShapeDtypeStruct((NC, NP, D), jnp.float32)]
    scratch = [
        pltpu.VMEM((K,), jnp.int32),
        pltpu.VMEM((K,), jnp.int32),
        pltpu.VMEM((K,), jnp.int32),
        pltpu.VMEM((K,), jnp.int32),
        pltpu.VMEM((K, D), jnp.float32),
        pltpu.VMEM((K, D), jnp.float32),
        pltpu.VMEM_SHARED((NP, D), jnp.float32),
        pltpu.SemaphoreType.DMA,
        pltpu.SemaphoreType.DMA,
        pltpu.SemaphoreType.DMA,
        pltpu.SemaphoreType.DMA,
    ]
    if with_cnt:
        out_type.append(jax.ShapeDtypeStruct((NW, N), jnp.float32))
        scratch.append(pltpu.VMEM((N,), jnp.float32))

    def body(src_hbm, dst_hbm, y_hbm, *rest):
        if with_cnt:
            (agg_out, cnt_out, ia_s, ia_d, ib_s, ib_d, rows_a, rows_b,
             agg_sh, sem_ga, sem_gb, sem_ia, sem_ib, cnt_v) = rest
        else:
            (agg_out, ia_s, ia_d, ib_s, ib_d, rows_a, rows_b,
             agg_sh, sem_ga, sem_gb, sem_ia, sem_ib) = rest
        c = lax.axis_index("c")
        s = lax.axis_index("s")
        wid = c * NS + s
        ebase = wid * EPW

        def start_idx(i, bs, bd, sem):
            off = ebase + i * K
            pltpu.async_copy(src_hbm.at[pl.ds(off, K)], bs, sem)
            pltpu.async_copy(dst_hbm.at[pl.ds(off, K)], bd, sem)

        def wait_idx(i, bs, bd, sem):
            off = ebase + i * K
            pltpu.make_async_copy(src_hbm.at[pl.ds(off, K)], bs, sem).wait()
            pltpu.make_async_copy(dst_hbm.at[pl.ds(off, K)], bd, sem).wait()

        def start_g(bs, rows, sem):
            pltpu.async_copy(y_hbm.at[bs], rows, sem)

        def wait_g(bs, rows, sem):
            pltpu.make_async_copy(y_hbm.at[bs], rows, sem).wait()

        start_idx(0, ia_s, ia_d, sem_ia)
        start_idx(1, ib_s, ib_d, sem_ib)

        z16 = jnp.zeros((16,), jnp.float32)

        def zrow(r, carry):
            for g in range(D // 16):
                rows_a[r, pl.ds(g * 16, 16)] = z16
            return carry

        lax.fori_loop(0, K, zrow, 0)
        for k in range(RPT // K):
            pltpu.sync_copy(rows_a, agg_sh.at[pl.ds(s * RPT + k * K, K)])
        if with_cnt:
            def zcnt(t, carry):
                cnt_v[pl.ds(t * 16, 16)] = z16
                return carry

            lax.fori_loop(0, N // 16, zcnt, 0)

        ones16 = jnp.ones((16,), jnp.float32)

        def scat(bd, rows):
            if with_cnt:
                for g in range(K // 16):
                    plsc.addupdate_scatter(cnt_v, [bd[pl.ds(g * 16, 16)]],
                                           ones16)
            pltpu.sync_copy(rows, agg_sh.at[bd], add=True)

        wait_idx(0, ia_s, ia_d, sem_ia)
        start_g(ia_s, rows_a, sem_ga)
        plsc.subcore_barrier()

        last = NCHUNK - 1

        def pair(j, carry):
            i1 = 2 * j + 1
            wait_g(ia_s, rows_a, sem_ga)
            wait_idx(i1, ib_s, ib_d, sem_ib)
            start_g(ib_s, rows_b, sem_gb)
            scat(ia_d, rows_a)
            start_idx(i1 + 1, ia_s, ia_d, sem_ia)
            wait_g(ib_s, rows_b, sem_gb)
            wait_idx(i1 + 1, ia_s, ia_d, sem_ia)
            start_g(ia_s, rows_a, sem_ga)
            scat(ib_d, rows_b)
            start_idx(jnp.minimum(i1 + 2, last), ib_s, ib_d, sem_ib)
            return carry

        lax.fori_loop(0, NPAIR, pair, 0)
        wait_idx(last, ib_s, ib_d, sem_ib)
        wait_g(ia_s, rows_a, sem_ga)
        scat(ia_d, rows_a)
        plsc.subcore_barrier()

        sl = pl.ds(s * RPT, RPT)
        pltpu.sync_copy(agg_sh.at[sl], agg_out.at[c, sl])
        if with_cnt:
            pltpu.sync_copy(cnt_v, cnt_out.at[wid])

    ot = out_type if with_cnt else out_type[0]
    mesh = plsc.VectorSubcoreMesh(core_axis_name="c", subcore_axis_name="s")
    return pl.kernel(
        body, mesh=mesh, out_type=ot, scratch_types=scratch,
        compiler_params=pltpu.CompilerParams(use_tc_tiling_on_sc=False,
                                             needs_layout_passes=False))


BR = 1000


def _ln_relu(x, g, b):
    m = jnp.mean(x, axis=-1, keepdims=True)
    v = jnp.mean((x - m) ** 2, axis=-1, keepdims=True)
    h = (x - m) * lax.rsqrt(v + 1e-5) * g + b
    return jnp.maximum(h, 0.0)


def _tc_pre_body(x_ref, g_ref, b_ref, wlt_ref, wrt_ref, y_ref, s_ref):
    h = _ln_relu(x_ref[...], g_ref[...], b_ref[...])
    y_ref[...] = jnp.dot(h, wlt_ref[...], preferred_element_type=jnp.float32)
    s_ref[...] = jnp.dot(h, wrt_ref[...], preferred_element_type=jnp.float32)


def _mean_from_partials(agg, cnt):
    a = agg[0] + agg[1]
    c = jnp.sum(cnt, axis=-1, keepdims=True)
    return a / jnp.maximum(c, 1.0)


def _tc_mid_body(agg_ref, cnt_ref, s0_ref, b0l_ref, g_ref, b_ref,
                 wlt_ref, wrt_ref, y_ref, s_ref):
    t = _mean_from_partials(agg_ref[...], cnt_ref[...]) + b0l_ref[...] + s0_ref[...]
    h = _ln_relu(t, g_ref[...], b_ref[...])
    y_ref[...] = jnp.dot(h, wlt_ref[...], preferred_element_type=jnp.float32)
    s_ref[...] = jnp.dot(h, wrt_ref[...], preferred_element_type=jnp.float32)


def _tc_post_body(x_ref, agg_ref, cnt_ref, s1_ref, b1l_ref, out_ref):
    t = _mean_from_partials(agg_ref[...], cnt_ref[...]) + b1l_ref[...] + s1_ref[...]
    out_ref[...] = x_ref[...] + t


_row_spec = pl.BlockSpec((BR, D), lambda i: (i, 0))
_vec_spec = pl.BlockSpec((1, D), lambda i: (0, 0))
_w_spec = pl.BlockSpec((D, D), lambda i: (0, 0))
_agg_spec = pl.BlockSpec((NC, BR, D), lambda i: (0, i, 0))
_cnt_spec = pl.BlockSpec((BR, NW), lambda i: (i, 0))
_GRID = (N // BR,)

_tc_pre = pl.pallas_call(
    _tc_pre_body,
    grid=_GRID,
    in_specs=[_row_spec, _vec_spec, _vec_spec, _w_spec, _w_spec],
    out_specs=[_row_spec, _row_spec],
    out_shape=[jax.ShapeDtypeStruct((N, D), jnp.float32)] * 2,
)

_tc_mid = pl.pallas_call(
    _tc_mid_body,
    grid=_GRID,
    in_specs=[_agg_spec, _cnt_spec, _row_spec, _vec_spec, _vec_spec,
              _vec_spec, _w_spec, _w_spec],
    out_specs=[_row_spec, _row_spec],
    out_shape=[jax.ShapeDtypeStruct((N, D), jnp.float32)] * 2,
)

_tc_post = pl.pallas_call(
    _tc_post_body,
    grid=_GRID,
    in_specs=[_row_spec, _agg_spec, _cnt_spec, _row_spec, _vec_spec],
    out_specs=_row_spec,
    out_shape=jax.ShapeDtypeStruct((N, D), jnp.float32),
)


def kernel(x, edge_index, ln0_g, ln0_b, w0l, b0l, w0r, ln1_g, ln1_b,
           w1l, b1l, w1r):
    src = edge_index[0]
    dst = edge_index[1]
    g0 = ln0_g.reshape(1, D)
    b0 = ln0_b.reshape(1, D)
    g1 = ln1_g.reshape(1, D)
    b1 = ln1_b.reshape(1, D)
    b0l2 = b0l.reshape(1, D)
    b1l2 = b1l.reshape(1, D)
    y0, s0 = _tc_pre(x, g0, b0, w0l.T, w0r.T)
    agg0, cnt_p = _make_sc_agg(True)(src, dst, y0)
    cnt = cnt_p.T
    y1, s1 = _tc_mid(agg0, cnt, s0, b0l2, g1, b1, w1l.T, w1r.T)
    agg1 = _make_sc_agg(False)(src, dst, y1)
    return _tc_post(x, agg1, cnt, s1, b1l2)
